# Initial kernel scaffold; baseline (speedup 1.0000x reference)
import jax
import jax.numpy as jnp
from jax.experimental import pallas as pl


def kernel(x_seq, params):
    raise NotImplementedError("write your pallas kernel here")


# speedup vs baseline: 2.3592x; 2.3592x over previous
import jax
import jax.numpy as jnp
from jax.experimental import pallas as pl
from jax.experimental.pallas import tpu as pltpu

B = 8
SEQ_LEN = 512
N_VARS = 512
D_MODEL = 1024
N_HEADS = 16
D_HEAD = D_MODEL // N_HEADS
D_FF = 16
N_LAYERS = 4
N_EXPERTS = 8
OUT_LEN = 96
EPS = 1e-5

_INTERPRET = False


def _dot(a, b, dims=None, precision=None):
    if dims is None:
        dims = ([a.ndim - 1], [0])
    return jax.lax.dot_general(a, b, (dims, ((), ())),
                               preferred_element_type=jnp.float32,
                               precision=precision)


def _ln(x, w, b):
    m = jnp.mean(x, axis=-1, keepdims=True)
    v = jnp.mean((x - m) ** 2, axis=-1, keepdims=True)
    return (x - m) * (w / jnp.sqrt(v + EPS)) + b


def _top2_weights(g):
    iota = jax.lax.broadcasted_iota(jnp.int32, g.shape, 1)
    m1 = jnp.max(g, axis=1, keepdims=True)
    i1 = jnp.min(jnp.where(g == m1, iota, N_EXPERTS), axis=1, keepdims=True)
    mask1 = iota == i1
    gm = jnp.where(mask1, -jnp.inf, g)
    m2 = jnp.max(gm, axis=1, keepdims=True)
    i2 = jnp.min(jnp.where(gm == m2, iota, N_EXPERTS), axis=1, keepdims=True)
    return jnp.where(mask1 | (iota == i2), g, 0.0)


def _softmax(x):
    e = jnp.exp(x - jnp.max(x, axis=-1, keepdims=True))
    return e / jnp.sum(e, axis=-1, keepdims=True)


def _revin_embed_kernel(x_ref, rw_ref, rb_ref, eW_ref, eb_ref,
                        enc_ref, mean_ref, std_ref):
    x = x_ref[0]
    m = jnp.mean(x, axis=0, keepdims=True)
    v = jnp.mean((x - m) ** 2, axis=0, keepdims=True)
    s = jnp.sqrt(v + EPS)
    xn = (x - m) * (rw_ref[...] / s) + rb_ref[...]
    enc = _dot(xn, eW_ref[...], dims=([0], [0])) + eb_ref[...]
    enc_ref[0] = enc
    mean_ref[0] = m
    std_ref[0] = s


def _layer_kernel(enc_ref, qW_ref, qb_ref, kW_ref, kb_ref, vW_ref, vb_ref,
                  oW_ref, ob_ref, n1w_ref, n1b_ref, n2w_ref, n2b_ref,
                  gW_ref, W1_ref, b1_ref, W2_ref, b2_ref,
                  out_ref, gates_ref):
    x = enc_ref[0]
    q = _dot(x, qW_ref[...]) + qb_ref[...]
    k = _dot(x, kW_ref[...]) + kb_ref[...]
    v = _dot(x, vW_ref[...]) + vb_ref[...]
    scale = 1.0 / (D_HEAD ** 0.5)
    outs = []
    for h in range(N_HEADS):
        sl = slice(h * D_HEAD, (h + 1) * D_HEAD)
        qh, kh, vh = q[:, sl], k[:, sl], v[:, sl]
        s = _dot(qh, kh, dims=([1], [1])) * scale
        a = _softmax(s)
        outs.append(_dot(a, vh))
    o = jnp.concatenate(outs, axis=1)
    attn = _dot(o, oW_ref[...]) + ob_ref[...]
    y = _ln(x + attn, n1w_ref[...], n1b_ref[...])

    g = _softmax(_dot(y, gW_ref[...]))
    w2 = _top2_weights(g)
    h_ = jnp.maximum(_dot(y, W1_ref[...]) + b1_ref[...], 0.0)
    moe = jnp.zeros((N_VARS, D_MODEL), jnp.float32)
    for i in range(N_EXPERTS):
        eo = _dot(h_[:, i * D_FF:(i + 1) * D_FF],
                  W2_ref[i * D_FF:(i + 1) * D_FF, :]) + b2_ref[i:i + 1, :]
        moe = moe + w2[:, i:i + 1] * eo
    out_ref[0] = _ln(y + moe, n2w_ref[...], n2b_ref[...])

    b = pl.program_id(0)

    @pl.when(b == 0)
    def _():
        gates_ref[...] = g * (1.0 / B)

    @pl.when(b != 0)
    def _():
        gates_ref[...] += g * (1.0 / B)


def _head_kernel(enc_ref, fnw_ref, fnb_ref, gW_ref, W_ref, b_ref,
                 mean_ref, std_ref, rw_ref, rb_ref, out_ref):
    x = _ln(enc_ref[0], fnw_ref[...], fnb_ref[...])
    g = _softmax(_dot(x, gW_ref[...]))
    w2 = _top2_weights(g)
    H = _dot(x, W_ref[...])
    o = jnp.zeros((N_VARS, OUT_LEN), jnp.float32)
    for i in range(N_EXPERTS):
        o = o + w2[:, i:i + 1] * (H[:, i * OUT_LEN:(i + 1) * OUT_LEN]
                                  + b_ref[i:i + 1, :])
    oT = jnp.transpose(o)
    mean = mean_ref[0]
    std = std_ref[0]
    out_ref[0] = (oT - rb_ref[...]) * (std / (rw_ref[...] + EPS * EPS)) + mean


def _full(shape):
    n = len(shape)
    return pl.BlockSpec(shape, lambda b: (0,) * n)


def kernel(x_seq, params):
    f32 = jnp.float32
    p = params
    row = lambda a: a.reshape(1, -1)

    grid = (B,)
    cp = pltpu.CompilerParams(dimension_semantics=("arbitrary",))

    enc, mean, std = pl.pallas_call(
        _revin_embed_kernel,
        grid=grid,
        in_specs=[
            pl.BlockSpec((1, SEQ_LEN, N_VARS), lambda b: (b, 0, 0)),
            _full((1, N_VARS)), _full((1, N_VARS)),
            _full((SEQ_LEN, D_MODEL)), _full((1, D_MODEL)),
        ],
        out_specs=[
            pl.BlockSpec((1, N_VARS, D_MODEL), lambda b: (b, 0, 0)),
            pl.BlockSpec((1, 1, N_VARS), lambda b: (b, 0, 0)),
            pl.BlockSpec((1, 1, N_VARS), lambda b: (b, 0, 0)),
        ],
        out_shape=[
            jax.ShapeDtypeStruct((B, N_VARS, D_MODEL), f32),
            jax.ShapeDtypeStruct((B, 1, N_VARS), f32),
            jax.ShapeDtypeStruct((B, 1, N_VARS), f32),
        ],
        compiler_params=cp,
        interpret=_INTERPRET,
    )(x_seq, row(p['revin_w']), row(p['revin_b']), p['emb_W'], row(p['emb_b']))

    gate_list = []
    for lp in p['layers']:
        W1 = jnp.concatenate([e['W1'] for e in lp['experts']], axis=1)
        b1 = jnp.concatenate([e['b1'] for e in lp['experts']]).reshape(1, -1)
        W2 = jnp.concatenate([e['W2'] for e in lp['experts']], axis=0)
        b2 = jnp.stack([e['b2'] for e in lp['experts']])
        enc, gates = pl.pallas_call(
            _layer_kernel,
            grid=grid,
            in_specs=[
                pl.BlockSpec((1, N_VARS, D_MODEL), lambda b: (b, 0, 0)),
                _full((D_MODEL, D_MODEL)), _full((1, D_MODEL)),
                _full((D_MODEL, D_MODEL)), _full((1, D_MODEL)),
                _full((D_MODEL, D_MODEL)), _full((1, D_MODEL)),
                _full((D_MODEL, D_MODEL)), _full((1, D_MODEL)),
                _full((1, D_MODEL)), _full((1, D_MODEL)),
                _full((1, D_MODEL)), _full((1, D_MODEL)),
                _full((D_MODEL, N_EXPERTS)),
                _full((D_MODEL, N_EXPERTS * D_FF)), _full((1, N_EXPERTS * D_FF)),
                _full((N_EXPERTS * D_FF, D_MODEL)), _full((N_EXPERTS, D_MODEL)),
            ],
            out_specs=[
                pl.BlockSpec((1, N_VARS, D_MODEL), lambda b: (b, 0, 0)),
                pl.BlockSpec((N_VARS, N_EXPERTS), lambda b: (0, 0)),
            ],
            out_shape=[
                jax.ShapeDtypeStruct((B, N_VARS, D_MODEL), f32),
                jax.ShapeDtypeStruct((N_VARS, N_EXPERTS), f32),
            ],
            compiler_params=cp,
            interpret=_INTERPRET,
        )(enc, lp['qW'], row(lp['qb']), lp['kW'], row(lp['kb']),
          lp['vW'], row(lp['vb']), lp['oW'], row(lp['ob']),
          row(lp['n1w']), row(lp['n1b']), row(lp['n2w']), row(lp['n2b']),
          lp['gW'], W1, b1, W2, b2)
        gate_list.append(gates)

    HW = jnp.concatenate([e['W'] for e in p['head_experts']], axis=1)
    Hb = jnp.stack([e['b'] for e in p['head_experts']])
    out = pl.pallas_call(
        _head_kernel,
        grid=grid,
        in_specs=[
            pl.BlockSpec((1, N_VARS, D_MODEL), lambda b: (b, 0, 0)),
            _full((1, D_MODEL)), _full((1, D_MODEL)),
            _full((D_MODEL, N_EXPERTS)),
            _full((D_MODEL, N_EXPERTS * OUT_LEN)), _full((N_EXPERTS, OUT_LEN)),
            pl.BlockSpec((1, 1, N_VARS), lambda b: (b, 0, 0)),
            pl.BlockSpec((1, 1, N_VARS), lambda b: (b, 0, 0)),
            _full((1, N_VARS)), _full((1, N_VARS)),
        ],
        out_specs=pl.BlockSpec((1, OUT_LEN, N_VARS), lambda b: (b, 0, 0)),
        out_shape=jax.ShapeDtypeStruct((B, OUT_LEN, N_VARS), f32),
        compiler_params=cp,
        interpret=_INTERPRET,
    )(enc, row(p['final_nw']), row(p['final_nb']), p['head_gW'], HW, Hb,
      mean, std, row(p['revin_w']), row(p['revin_b']))

    return out, jnp.stack(gate_list)
